# Initial kernel scaffold; baseline (speedup 1.0000x reference)
import jax
import jax.numpy as jnp
from jax.experimental import pallas as pl


def kernel(x, edge_index, edge_attr, batch, W_f, b_f, W_s, b_s, W_out, b_out):
    raise NotImplementedError("write your pallas kernel here")


# speedup vs baseline: 3.0730x; 3.0730x over previous
import functools

import jax
import jax.numpy as jnp
from jax import lax
from jax.experimental import pallas as pl
from jax.experimental.pallas import tpu as pltpu
from jax.experimental.pallas import tpu_sc as plsc

_N, _E, _F, _DE, _G = 10000, 320000, 128, 16, 64

_NC, _NS = 2, 16
_NW = _NC * _NS
_EPW = _E // _NW
_CH = 80
_NCHUNK = _EPW // _CH

_BN = 2000
_BE = 2000
_NBN = _N // _BN
_NBE = _E // _BE


def _proj_body(x_ref, wd_ref, ws_ref, wo_ref, batch_ref, pd_ref, ps_ref, nb_ref):
    x = x_ref[...]
    pd_ref[...] = jnp.dot(x, wd_ref[...], preferred_element_type=jnp.float32)
    ps_ref[...] = jnp.dot(x, ws_ref[...], preferred_element_type=jnp.float32)
    xw = jnp.dot(x, wo_ref[...], preferred_element_type=jnp.float32)
    b = batch_ref[0, 0, :]
    mask = b[:, None] == lax.broadcasted_iota(jnp.int32, (1, _G), 1)
    part = jnp.sum(jnp.where(mask, xw, 0.0), axis=0)

    @pl.when(pl.program_id(0) == 0)
    def _():
        nb_ref[...] = jnp.zeros_like(nb_ref)

    nb_ref[...] += jnp.broadcast_to(part[None, :], nb_ref.shape)


def _proj_call(x_aug, wd_aug, ws_aug, wo_aug, batch3):
    return pl.pallas_call(
        _proj_body,
        grid=(_NBN,),
        in_specs=[
            pl.BlockSpec((_BN, _F + 1), lambda i: (i, 0)),
            pl.BlockSpec((_F + 1, 2 * _F), lambda i: (0, 0)),
            pl.BlockSpec((_F + 1, 2 * _F), lambda i: (0, 0)),
            pl.BlockSpec((_F + 1, 1), lambda i: (0, 0)),
            pl.BlockSpec((1, 1, _BN), lambda i: (i, 0, 0)),
        ],
        out_specs=[
            pl.BlockSpec((_BN, 2 * _F), lambda i: (i, 0)),
            pl.BlockSpec((_BN, 2 * _F), lambda i: (i, 0)),
            pl.BlockSpec((8, _G), lambda i: (0, 0)),
        ],
        out_shape=[
            jax.ShapeDtypeStruct((_N, 2 * _F), jnp.float32),
            jax.ShapeDtypeStruct((_N, 2 * _F), jnp.float32),
            jax.ShapeDtypeStruct((8, _G), jnp.float32),
        ],
    )(x_aug, wd_aug, ws_aug, wo_aug, batch3)


def _sc_gather_body(pd_hbm, ps_hbm, src_hbm, dst_hbm, batch_hbm,
                    ad_hbm, as_hbm, g_hbm,
                    dstv, srcv, rowsd, rowss, gv, semd, sems, semg):
    wid = lax.axis_index("s") * _NC + lax.axis_index("c")
    base = wid * _EPW

    def chunk(i, carry):
        off = base + i * _CH
        pltpu.sync_copy(dst_hbm.at[pl.ds(off, _CH)], dstv)
        pltpu.sync_copy(src_hbm.at[pl.ds(off, _CH)], srcv)
        cd = pltpu.async_copy(pd_hbm.at[dstv], rowsd, semd)
        cs = pltpu.async_copy(ps_hbm.at[srcv], rowss, sems)
        cg = pltpu.async_copy(batch_hbm.at[dstv], gv, semg)
        cd.wait()
        cs.wait()
        cg.wait()
        pltpu.sync_copy(rowsd, ad_hbm.at[pl.ds(off, _CH)])
        pltpu.sync_copy(rowss, as_hbm.at[pl.ds(off, _CH)])
        pltpu.sync_copy(gv, g_hbm.at[pl.ds(off, _CH)])
        return carry

    lax.fori_loop(0, _NCHUNK, chunk, 0)


def _sc_gather(pd, ps, src, dst, batch):
    mesh = plsc.VectorSubcoreMesh(
        core_axis_name="c", subcore_axis_name="s",
        num_cores=_NC, num_subcores=_NS)
    f = pl.kernel(
        _sc_gather_body,
        out_type=[
            jax.ShapeDtypeStruct((_E, 2 * _F), jnp.float32),
            jax.ShapeDtypeStruct((_E, 2 * _F), jnp.float32),
            jax.ShapeDtypeStruct((_E,), jnp.int32),
        ],
        mesh=mesh,
        scratch_types=[
            pltpu.VMEM((_CH,), jnp.int32),
            pltpu.VMEM((_CH,), jnp.int32),
            pltpu.VMEM((_CH, 2 * _F), jnp.float32),
            pltpu.VMEM((_CH, 2 * _F), jnp.float32),
            pltpu.VMEM((_CH,), jnp.int32),
            pltpu.SemaphoreType.DMA,
            pltpu.SemaphoreType.DMA,
            pltpu.SemaphoreType.DMA,
        ],
    )
    return f(pd, ps, src, dst, batch)


def _edge_body(ad_ref, as_ref, ea_ref, g_ref, we_ref, wo_ref, eb_ref):
    t = ad_ref[...] + as_ref[...] + jnp.dot(
        ea_ref[...], we_ref[...], preferred_element_type=jnp.float32)
    gate = jax.nn.sigmoid(t[:, :_F])
    z = t[:, _F:]
    core = jnp.maximum(z, 0.0) + jnp.log1p(jnp.exp(-jnp.abs(z)))
    s = jnp.dot(gate * core, wo_ref[...],
                preferred_element_type=jnp.float32)
    g = g_ref[0, 0, :]
    mask = g[:, None] == lax.broadcasted_iota(jnp.int32, (1, _G), 1)
    part = jnp.sum(jnp.where(mask, s, 0.0), axis=0)

    @pl.when(pl.program_id(0) == 0)
    def _():
        eb_ref[...] = jnp.zeros_like(eb_ref)

    eb_ref[...] += jnp.broadcast_to(part[None, :], eb_ref.shape)


def _edge_call(ad, as_, ea, g3, we, wo):
    return pl.pallas_call(
        _edge_body,
        grid=(_NBE,),
        in_specs=[
            pl.BlockSpec((_BE, 2 * _F), lambda i: (i, 0)),
            pl.BlockSpec((_BE, 2 * _F), lambda i: (i, 0)),
            pl.BlockSpec((_BE, _DE), lambda i: (i, 0)),
            pl.BlockSpec((1, 1, _BE), lambda i: (i, 0, 0)),
            pl.BlockSpec((_DE, 2 * _F), lambda i: (0, 0)),
            pl.BlockSpec((_F, 1), lambda i: (0, 0)),
        ],
        out_specs=pl.BlockSpec((8, _G), lambda i: (0, 0)),
        out_shape=jax.ShapeDtypeStruct((8, _G), jnp.float32),
    )(ad, as_, ea, g3, we, wo)


def kernel(x, edge_index, edge_attr, batch, W_f, b_f, W_s, b_s, W_out, b_out):
    f = _F
    src = edge_index[0]
    dst = edge_index[1]
    x_aug = jnp.concatenate([x, jnp.ones((_N, 1), jnp.float32)], axis=1)
    wd = jnp.concatenate([W_f[:f], W_s[:f]], axis=1)
    wsrc = jnp.concatenate([W_f[f:2 * f], W_s[f:2 * f]], axis=1)
    bcat = jnp.concatenate([b_f, b_s])[None, :]
    wd_aug = jnp.concatenate([wd, bcat], axis=0)
    wsrc_aug = jnp.concatenate(
        [wsrc, jnp.zeros((1, 2 * f), jnp.float32)], axis=0)
    wo_aug = jnp.concatenate([W_out, jnp.zeros((1, 1), jnp.float32)], axis=0)
    we = jnp.concatenate([W_f[2 * f:], W_s[2 * f:]], axis=1)

    batch3 = batch.reshape(_NBN, 1, _BN)
    pd, ps, nb = _proj_call(x_aug, wd_aug, wsrc_aug, wo_aug, batch3)

    ad, as_, g = _sc_gather(pd, ps, src, dst, batch)

    g3 = g.reshape(_NBE, 1, _BE)
    eb = _edge_call(ad, as_, edge_attr, g3, we, W_out)

    pooled_w = nb[0] + eb[0]
    return pooled_w[:, None] + b_out[None, :]


# speedup vs baseline: 4.0860x; 1.3296x over previous
import functools

import jax
import jax.numpy as jnp
from jax import lax
from jax.experimental import pallas as pl
from jax.experimental.pallas import tpu as pltpu
from jax.experimental.pallas import tpu_sc as plsc

_N, _E, _F, _DE, _G = 10000, 320000, 128, 16, 64

_NC, _NS = 2, 16
_NW = _NC * _NS
_EPW = _E // _NW
_CH = 80
_NCHUNK = _EPW // _CH

_BN = 2000
_BE = 2000
_NBN = _N // _BN
_NBE = _E // _BE


def _pack2(gate_f32, core_f32):
    glo = lax.bitcast_convert_type(
        gate_f32.astype(jnp.bfloat16), jnp.uint16).astype(jnp.uint32)
    chi = lax.bitcast_convert_type(
        core_f32.astype(jnp.bfloat16), jnp.uint16).astype(jnp.uint32)
    return lax.bitcast_convert_type((chi << 16) | glo, jnp.int32)


def _unpack_lo(w_u32):
    return lax.bitcast_convert_type(w_u32 << 16, jnp.float32)


def _unpack_hi(w_u32):
    return lax.bitcast_convert_type(w_u32 & jnp.uint32(0xFFFF0000), jnp.float32)


def _proj_body(x_ref, wd_ref, ws_ref, wo_ref, batch_ref, pd_ref, ps_ref, nb_ref):
    x = x_ref[...]
    pd = jnp.dot(x, wd_ref[...], preferred_element_type=jnp.float32)
    ps = jnp.dot(x, ws_ref[...], preferred_element_type=jnp.float32)
    pd_ref[...] = _pack2(pd[:, :_F], pd[:, _F:])
    ps_ref[...] = _pack2(ps[:, :_F], ps[:, _F:])
    xw = jnp.dot(x, wo_ref[...], preferred_element_type=jnp.float32)
    b = batch_ref[0, 0, :]
    mask = b[:, None] == lax.broadcasted_iota(jnp.int32, (1, _G), 1)
    part = jnp.sum(jnp.where(mask, xw, 0.0), axis=0)

    @pl.when(pl.program_id(0) == 0)
    def _():
        nb_ref[...] = jnp.zeros_like(nb_ref)

    nb_ref[...] += jnp.broadcast_to(part[None, :], nb_ref.shape)


def _proj_call(x_aug, wd_aug, ws_aug, wo_aug, batch3):
    return pl.pallas_call(
        _proj_body,
        grid=(_NBN,),
        in_specs=[
            pl.BlockSpec((_BN, _F + 1), lambda i: (i, 0)),
            pl.BlockSpec((_F + 1, 2 * _F), lambda i: (0, 0)),
            pl.BlockSpec((_F + 1, 2 * _F), lambda i: (0, 0)),
            pl.BlockSpec((_F + 1, 1), lambda i: (0, 0)),
            pl.BlockSpec((1, 1, _BN), lambda i: (i, 0, 0)),
        ],
        out_specs=[
            pl.BlockSpec((_BN, _F), lambda i: (i, 0)),
            pl.BlockSpec((_BN, _F), lambda i: (i, 0)),
            pl.BlockSpec((8, _G), lambda i: (0, 0)),
        ],
        out_shape=[
            jax.ShapeDtypeStruct((_N, _F), jnp.int32),
            jax.ShapeDtypeStruct((_N, _F), jnp.int32),
            jax.ShapeDtypeStruct((8, _G), jnp.float32),
        ],
    )(x_aug, wd_aug, ws_aug, wo_aug, batch3)


def _sc_gather_body(pd_hbm, ps_hbm, src_hbm, dst_hbm, batch_hbm,
                    ad_hbm, as_hbm, g_hbm,
                    dstv, srcv, rowsd, rowss, gv, semd, sems, semg):
    wid = lax.axis_index("s") * _NC + lax.axis_index("c")
    base = wid * _EPW

    def chunk(i, carry):
        off = base + i * _CH
        pltpu.sync_copy(dst_hbm.at[pl.ds(off, _CH)], dstv)
        pltpu.sync_copy(src_hbm.at[pl.ds(off, _CH)], srcv)
        cd = pltpu.async_copy(pd_hbm.at[dstv], rowsd, semd)
        cs = pltpu.async_copy(ps_hbm.at[srcv], rowss, sems)
        cg = pltpu.async_copy(batch_hbm.at[dstv], gv, semg)
        cd.wait()
        cs.wait()
        cg.wait()
        pltpu.sync_copy(rowsd, ad_hbm.at[pl.ds(off, _CH)])
        pltpu.sync_copy(rowss, as_hbm.at[pl.ds(off, _CH)])
        pltpu.sync_copy(gv, g_hbm.at[pl.ds(off, _CH)])
        return carry

    lax.fori_loop(0, _NCHUNK, chunk, 0)


def _sc_gather(pd, ps, src, dst, batch):
    mesh = plsc.VectorSubcoreMesh(
        core_axis_name="c", subcore_axis_name="s",
        num_cores=_NC, num_subcores=_NS)
    f = pl.kernel(
        _sc_gather_body,
        out_type=[
            jax.ShapeDtypeStruct((_E, _F), jnp.int32),
            jax.ShapeDtypeStruct((_E, _F), jnp.int32),
            jax.ShapeDtypeStruct((_E,), jnp.int32),
        ],
        mesh=mesh,
        scratch_types=[
            pltpu.VMEM((_CH,), jnp.int32),
            pltpu.VMEM((_CH,), jnp.int32),
            pltpu.VMEM((_CH, _F), jnp.int32),
            pltpu.VMEM((_CH, _F), jnp.int32),
            pltpu.VMEM((_CH,), jnp.int32),
            pltpu.SemaphoreType.DMA,
            pltpu.SemaphoreType.DMA,
            pltpu.SemaphoreType.DMA,
        ],
    )
    return f(pd, ps, src, dst, batch)


def _edge_body(ad_ref, as_ref, ea_ref, g_ref, we_ref, wo_ref, eb_ref):
    q = jnp.dot(ea_ref[...], we_ref[...],
                preferred_element_type=jnp.float32)
    adu = lax.bitcast_convert_type(ad_ref[...], jnp.uint32)
    asu = lax.bitcast_convert_type(as_ref[...], jnp.uint32)
    tg = _unpack_lo(adu) + _unpack_lo(asu) + q[:, :_F]
    tc = _unpack_hi(adu) + _unpack_hi(asu) + q[:, _F:]
    gate = jax.nn.sigmoid(tg)
    core = jnp.maximum(tc, 0.0) + jnp.log1p(jnp.exp(-jnp.abs(tc)))
    s = jnp.dot(gate * core, wo_ref[...],
                preferred_element_type=jnp.float32)
    g = g_ref[0, 0, :]
    mask = g[:, None] == lax.broadcasted_iota(jnp.int32, (1, _G), 1)
    part = jnp.sum(jnp.where(mask, s, 0.0), axis=0)

    @pl.when(pl.program_id(0) == 0)
    def _():
        eb_ref[...] = jnp.zeros_like(eb_ref)

    eb_ref[...] += jnp.broadcast_to(part[None, :], eb_ref.shape)


def _edge_call(ad, as_, ea, g3, we, wo):
    return pl.pallas_call(
        _edge_body,
        grid=(_NBE,),
        in_specs=[
            pl.BlockSpec((_BE, _F), lambda i: (i, 0)),
            pl.BlockSpec((_BE, _F), lambda i: (i, 0)),
            pl.BlockSpec((_BE, _DE), lambda i: (i, 0)),
            pl.BlockSpec((1, 1, _BE), lambda i: (i, 0, 0)),
            pl.BlockSpec((_DE, 2 * _F), lambda i: (0, 0)),
            pl.BlockSpec((_F, 1), lambda i: (0, 0)),
        ],
        out_specs=pl.BlockSpec((8, _G), lambda i: (0, 0)),
        out_shape=jax.ShapeDtypeStruct((8, _G), jnp.float32),
    )(ad, as_, ea, g3, we, wo)


def kernel(x, edge_index, edge_attr, batch, W_f, b_f, W_s, b_s, W_out, b_out):
    f = _F
    src = edge_index[0]
    dst = edge_index[1]
    x_aug = jnp.concatenate([x, jnp.ones((_N, 1), jnp.float32)], axis=1)
    wd = jnp.concatenate([W_f[:f], W_s[:f]], axis=1)
    wsrc = jnp.concatenate([W_f[f:2 * f], W_s[f:2 * f]], axis=1)
    bcat = jnp.concatenate([b_f, b_s])[None, :]
    wd_aug = jnp.concatenate([wd, bcat], axis=0)
    wsrc_aug = jnp.concatenate(
        [wsrc, jnp.zeros((1, 2 * f), jnp.float32)], axis=0)
    wo_aug = jnp.concatenate([W_out, jnp.zeros((1, 1), jnp.float32)], axis=0)
    we = jnp.concatenate([W_f[2 * f:], W_s[2 * f:]], axis=1)

    batch3 = batch.reshape(_NBN, 1, _BN)
    pd, ps, nb = _proj_call(x_aug, wd_aug, wsrc_aug, wo_aug, batch3)

    ad, as_, g = _sc_gather(pd, ps, src, dst, batch)

    g3 = g.reshape(_NBE, 1, _BE)
    eb = _edge_call(ad, as_, edge_attr, g3, we, W_out)

    pooled_w = nb[0] + eb[0]
    return pooled_w[:, None] + b_out[None, :]


# speedup vs baseline: 4.5615x; 1.1164x over previous
import functools

import jax
import jax.numpy as jnp
from jax import lax
from jax.experimental import pallas as pl
from jax.experimental.pallas import tpu as pltpu
from jax.experimental.pallas import tpu_sc as plsc

_N, _E, _F, _DE, _G = 10000, 320000, 128, 16, 64

_NC, _NS = 2, 16
_NW = _NC * _NS
_EPW = _E // _NW
_CH = 80
_NCHUNK = _EPW // _CH

_BN = 2000
_BE = 2000
_NBN = _N // _BN
_NBE = _E // _BE


def _pack2(gate_f32, core_f32):
    glo = lax.bitcast_convert_type(
        gate_f32.astype(jnp.bfloat16), jnp.uint16).astype(jnp.uint32)
    chi = lax.bitcast_convert_type(
        core_f32.astype(jnp.bfloat16), jnp.uint16).astype(jnp.uint32)
    return lax.bitcast_convert_type((chi << 16) | glo, jnp.int32)


def _unpack_lo(w_u32):
    return lax.bitcast_convert_type(w_u32 << 16, jnp.float32)


def _unpack_hi(w_u32):
    return lax.bitcast_convert_type(w_u32 & jnp.uint32(0xFFFF0000), jnp.float32)


def _proj_body(x_ref, wd_ref, ws_ref, wo_ref, batch_ref, pd_ref, ps_ref, nb_ref):
    x = x_ref[...]
    pd = jnp.dot(x, wd_ref[...], preferred_element_type=jnp.float32)
    ps = jnp.dot(x, ws_ref[...], preferred_element_type=jnp.float32)
    pd_ref[...] = _pack2(pd[:, :_F], pd[:, _F:])
    ps_ref[...] = _pack2(ps[:, :_F], ps[:, _F:])
    xw = jnp.dot(x, wo_ref[...], preferred_element_type=jnp.float32)
    b = batch_ref[0, 0, :]
    onehot = (b[:, None] == lax.broadcasted_iota(jnp.int32, (1, _G), 1)
              ).astype(jnp.float32)
    part = lax.dot_general(xw, onehot, (((0,), (0,)), ((), ())),
                           precision=lax.Precision.HIGHEST,
                           preferred_element_type=jnp.float32)

    @pl.when(pl.program_id(0) == 0)
    def _():
        nb_ref[...] = jnp.zeros_like(nb_ref)

    nb_ref[...] += jnp.broadcast_to(part, nb_ref.shape)


def _proj_call(x_aug, wd_aug, ws_aug, wo_aug, batch3):
    return pl.pallas_call(
        _proj_body,
        grid=(_NBN,),
        in_specs=[
            pl.BlockSpec((_BN, _F + 1), lambda i: (i, 0)),
            pl.BlockSpec((_F + 1, 2 * _F), lambda i: (0, 0)),
            pl.BlockSpec((_F + 1, 2 * _F), lambda i: (0, 0)),
            pl.BlockSpec((_F + 1, 1), lambda i: (0, 0)),
            pl.BlockSpec((1, 1, _BN), lambda i: (i, 0, 0)),
        ],
        out_specs=[
            pl.BlockSpec((_BN, _F), lambda i: (i, 0)),
            pl.BlockSpec((_BN, _F), lambda i: (i, 0)),
            pl.BlockSpec((8, _G), lambda i: (0, 0)),
        ],
        out_shape=[
            jax.ShapeDtypeStruct((_N, _F), jnp.int32),
            jax.ShapeDtypeStruct((_N, _F), jnp.int32),
            jax.ShapeDtypeStruct((8, _G), jnp.float32),
        ],
    )(x_aug, wd_aug, ws_aug, wo_aug, batch3)


def _sc_gather_body(pd_hbm, ps_hbm, src_hbm, dst_hbm, batch_hbm,
                    ad_hbm, as_hbm, g_hbm,
                    dstv0, srcv0, gv0, dstv1, srcv1, gv1,
                    rd0, rs0, rd1, rs1,
                    semi0, semi1, semg0, semg1, semw0, semw1):
    wid = lax.axis_index("s") * _NC + lax.axis_index("c")
    base = wid * _EPW

    dstv = (dstv0, dstv1)
    srcv = (srcv0, srcv1)
    gv = (gv0, gv1)
    rd = (rd0, rd1)
    rs = (rs0, rs1)
    semi = (semi0, semi1)
    semg = (semg0, semg1)
    semw = (semw0, semw1)

    def fire_idx(i, b):
        off = base + i * _CH
        pltpu.async_copy(dst_hbm.at[pl.ds(off, _CH)], dstv[b], semi[b])
        pltpu.async_copy(src_hbm.at[pl.ds(off, _CH)], srcv[b], semi[b])

    def fire_gathers(b):
        pltpu.make_async_copy(dst_hbm.at[pl.ds(base, _CH)], dstv[b],
                              semi[b]).wait()
        pltpu.make_async_copy(dst_hbm.at[pl.ds(base, _CH)], srcv[b],
                              semi[b]).wait()
        pltpu.async_copy(pd_hbm.at[dstv[b]], rd[b], semg[b])
        pltpu.async_copy(ps_hbm.at[srcv[b]], rs[b], semg[b])
        pltpu.async_copy(batch_hbm.at[dstv[b]], gv[b], semg[b])

    def drain_gathers(b):
        pltpu.make_async_copy(pd_hbm.at[pl.ds(0, _CH)], rd[b], semg[b]).wait()
        pltpu.make_async_copy(pd_hbm.at[pl.ds(0, _CH)], rs[b], semg[b]).wait()
        pltpu.make_async_copy(dst_hbm.at[pl.ds(0, _CH)], gv[b], semg[b]).wait()

    def write(i, b):
        off = base + i * _CH
        pltpu.async_copy(rd[b], ad_hbm.at[pl.ds(off, _CH)], semw[b])
        pltpu.async_copy(rs[b], as_hbm.at[pl.ds(off, _CH)], semw[b])
        pltpu.async_copy(gv[b], g_hbm.at[pl.ds(off, _CH)], semw[b])

    def drain_writes(b):
        pltpu.make_async_copy(rd[b], ad_hbm.at[pl.ds(base, _CH)], semw[b]).wait()
        pltpu.make_async_copy(rs[b], as_hbm.at[pl.ds(base, _CH)], semw[b]).wait()
        pltpu.make_async_copy(gv[b], g_hbm.at[pl.ds(base, _CH)], semw[b]).wait()

    fire_idx(0, 0)
    fire_idx(1, 1)
    fire_gathers(0)
    fire_gathers(1)

    def pair(j, carry):
        c0 = 2 * j
        for b in (0, 1):
            drain_gathers(b)
            write(c0 + b, b)
            fire_idx(c0 + b + 2, b)
            drain_writes(b)
            fire_gathers(b)
        return carry

    lax.fori_loop(0, (_NCHUNK - 3) // 2, pair, 0)
    drain_gathers(0)
    write(_NCHUNK - 3, 0)
    fire_idx(_NCHUNK - 1, 0)
    drain_writes(0)
    fire_gathers(0)
    drain_gathers(1)
    write(_NCHUNK - 2, 1)
    drain_gathers(0)
    write(_NCHUNK - 1, 0)
    drain_writes(1)
    drain_writes(0)


def _sc_gather(pd, ps, src, dst, batch):
    mesh = plsc.VectorSubcoreMesh(
        core_axis_name="c", subcore_axis_name="s",
        num_cores=_NC, num_subcores=_NS)
    f = pl.kernel(
        _sc_gather_body,
        out_type=[
            jax.ShapeDtypeStruct((_E, _F), jnp.int32),
            jax.ShapeDtypeStruct((_E, _F), jnp.int32),
            jax.ShapeDtypeStruct((_E,), jnp.int32),
        ],
        mesh=mesh,
        scratch_types=[
            pltpu.VMEM((_CH,), jnp.int32),
            pltpu.VMEM((_CH,), jnp.int32),
            pltpu.VMEM((_CH,), jnp.int32),
            pltpu.VMEM((_CH,), jnp.int32),
            pltpu.VMEM((_CH,), jnp.int32),
            pltpu.VMEM((_CH,), jnp.int32),
            pltpu.VMEM((_CH, _F), jnp.int32),
            pltpu.VMEM((_CH, _F), jnp.int32),
            pltpu.VMEM((_CH, _F), jnp.int32),
            pltpu.VMEM((_CH, _F), jnp.int32),
            pltpu.SemaphoreType.DMA,
            pltpu.SemaphoreType.DMA,
            pltpu.SemaphoreType.DMA,
            pltpu.SemaphoreType.DMA,
            pltpu.SemaphoreType.DMA,
            pltpu.SemaphoreType.DMA,
        ],
    )
    return f(pd, ps, src, dst, batch)


def _edge_body(ad_ref, as_ref, ea_ref, g_ref, we_ref, wo_ref, eb_ref):
    q = jnp.dot(ea_ref[...], we_ref[...],
                preferred_element_type=jnp.float32)
    adu = lax.bitcast_convert_type(ad_ref[...], jnp.uint32)
    asu = lax.bitcast_convert_type(as_ref[...], jnp.uint32)
    tg = _unpack_lo(adu) + _unpack_lo(asu) + q[:, :_F]
    tc = _unpack_hi(adu) + _unpack_hi(asu) + q[:, _F:]
    gate = jax.nn.sigmoid(tg)
    core = jnp.maximum(tc, 0.0) + jnp.log1p(jnp.exp(-jnp.abs(tc)))
    s = jnp.dot(gate * core, wo_ref[...],
                preferred_element_type=jnp.float32)
    g = g_ref[0, 0, :]
    onehot = (g[:, None] == lax.broadcasted_iota(jnp.int32, (1, _G), 1)
              ).astype(jnp.float32)
    part = lax.dot_general(s, onehot, (((0,), (0,)), ((), ())),
                           precision=lax.Precision.HIGHEST,
                           preferred_element_type=jnp.float32)

    @pl.when(pl.program_id(0) == 0)
    def _():
        eb_ref[...] = jnp.zeros_like(eb_ref)

    eb_ref[...] += jnp.broadcast_to(part, eb_ref.shape)


def _edge_call(ad, as_, ea, g3, we, wo):
    return pl.pallas_call(
        _edge_body,
        grid=(_NBE,),
        in_specs=[
            pl.BlockSpec((_BE, _F), lambda i: (i, 0)),
            pl.BlockSpec((_BE, _F), lambda i: (i, 0)),
            pl.BlockSpec((_BE, _DE), lambda i: (i, 0)),
            pl.BlockSpec((1, 1, _BE), lambda i: (i, 0, 0)),
            pl.BlockSpec((_DE, 2 * _F), lambda i: (0, 0)),
            pl.BlockSpec((_F, 1), lambda i: (0, 0)),
        ],
        out_specs=pl.BlockSpec((8, _G), lambda i: (0, 0)),
        out_shape=jax.ShapeDtypeStruct((8, _G), jnp.float32),
    )(ad, as_, ea, g3, we, wo)


def kernel(x, edge_index, edge_attr, batch, W_f, b_f, W_s, b_s, W_out, b_out):
    f = _F
    src = edge_index[0]
    dst = edge_index[1]
    x_aug = jnp.concatenate([x, jnp.ones((_N, 1), jnp.float32)], axis=1)
    wd = jnp.concatenate([W_f[:f], W_s[:f]], axis=1)
    wsrc = jnp.concatenate([W_f[f:2 * f], W_s[f:2 * f]], axis=1)
    bcat = jnp.concatenate([b_f, b_s])[None, :]
    wd_aug = jnp.concatenate([wd, bcat], axis=0)
    wsrc_aug = jnp.concatenate(
        [wsrc, jnp.zeros((1, 2 * f), jnp.float32)], axis=0)
    wo_aug = jnp.concatenate([W_out, jnp.zeros((1, 1), jnp.float32)], axis=0)
    we = jnp.concatenate([W_f[2 * f:], W_s[2 * f:]], axis=1)

    batch3 = batch.reshape(_NBN, 1, _BN)
    pd, ps, nb = _proj_call(x_aug, wd_aug, wsrc_aug, wo_aug, batch3)

    ad, as_, g = _sc_gather(pd, ps, src, dst, batch)

    g3 = g.reshape(_NBE, 1, _BE)
    eb = _edge_call(ad, as_, edge_attr, g3, we, W_out)

    pooled_w = nb[0] + eb[0]
    return pooled_w[:, None] + b_out[None, :]
